# Initial kernel scaffold; baseline (speedup 1.0000x reference)
import jax
import jax.numpy as jnp
from jax.experimental import pallas as pl


def kernel(query_residue_index, key_residue_index, embedding_weight):
    raise NotImplementedError("write your pallas kernel here")


# speedup vs baseline: 4.2322x; 4.2322x over previous
import functools

import jax
import jax.numpy as jnp
from jax import lax
from jax.experimental import pallas as pl
from jax.experimental.pallas import tpu as pltpu
from jax.experimental.pallas import tpu_sc as plsc

_BINS = 32
_EMBED = 16
_BATCH = 16384
_L = 200
_NUM_EMB = 2 * _BINS + 2

_NUM_CORES = 2
_NUM_SUBCORES = 16
_NUM_WORKERS = _NUM_CORES * _NUM_SUBCORES
_ROWS_PER_W = _BATCH // _NUM_WORKERS
_CB = 16
_CHUNKS = _ROWS_PER_W // _CB
_PAIRS = _CB * _L
_GROUPS = _PAIRS // 16

_DIV_MAGIC = 5243
_DIV_SHIFT = 20


def _sc_body(q_hbm, key_hbm, tab_hbm, out_hbm, tab_v, q_v, key_v, rows_v):
    wid = lax.axis_index("s") * _NUM_CORES + lax.axis_index("c")
    pltpu.sync_copy(tab_hbm, tab_v)
    iota = lax.iota(jnp.int32, 16)

    def chunk_body(c, carry):
        b0 = wid * _ROWS_PER_W + c * _CB
        p0 = b0 * _L
        pltpu.sync_copy(key_hbm.at[pl.ds(p0, _PAIRS)], key_v)
        pltpu.sync_copy(q_hbm.at[pl.ds(b0, _CB)], q_v)

        def group(v, inner):
            l16 = v * 16 + iota
            b_local = (l16 * _DIV_MAGIC) >> _DIV_SHIFT
            qv = plsc.load_gather(q_v, [b_local])
            kv = key_v[pl.ds(v * 16, 16)]
            idx = jnp.clip(kv - qv, -_BINS, _BINS) + (_BINS + 1)
            for e in range(_EMBED):
                es = jnp.full((16,), e, jnp.int32)
                col = plsc.load_gather(tab_v, [idx, es])
                plsc.store_scatter(rows_v, [l16, es], col)
            return inner

        lax.fori_loop(0, _GROUPS, group, 0)
        pltpu.sync_copy(rows_v, out_hbm.at[pl.ds(p0, _PAIRS)])
        return carry

    lax.fori_loop(0, _CHUNKS, chunk_body, 0)


@functools.partial(jax.jit, static_argnames=())
def kernel(query_residue_index, key_residue_index, embedding_weight):
    q = query_residue_index.astype(jnp.int32)
    k = key_residue_index.astype(jnp.int32).reshape(_BATCH * _L)
    w = embedding_weight.astype(jnp.float32)
    f = pl.kernel(
        _sc_body,
        mesh=plsc.VectorSubcoreMesh(core_axis_name="c", subcore_axis_name="s"),
        compiler_params=pltpu.CompilerParams(
            needs_layout_passes=False, use_tc_tiling_on_sc=False
        ),
        out_type=jax.ShapeDtypeStruct((_BATCH * _L, _EMBED), jnp.float32),
        scratch_types=[
            pltpu.VMEM((_NUM_EMB, _EMBED), jnp.float32),
            pltpu.VMEM((_CB,), jnp.int32),
            pltpu.VMEM((_PAIRS,), jnp.int32),
            pltpu.VMEM((_PAIRS, _EMBED), jnp.float32),
        ],
    )
    out = f(q, k, w)
    return out.reshape(_BATCH, _L, _EMBED)


# speedup vs baseline: 4.5377x; 1.0722x over previous
import functools

import jax
import jax.numpy as jnp
from jax import lax
from jax.experimental import pallas as pl
from jax.experimental.pallas import tpu as pltpu
from jax.experimental.pallas import tpu_sc as plsc

_BINS = 32
_EMBED = 16
_BATCH = 16384
_L = 200
_NUM_EMB = 2 * _BINS + 2

_NUM_CORES = 2
_NUM_SUBCORES = 16
_NUM_WORKERS = _NUM_CORES * _NUM_SUBCORES
_ROWS_PER_W = _BATCH // _NUM_WORKERS
_CB = 16
_CHUNKS = _ROWS_PER_W // _CB
_PAIRS = _CB * _L
_GROUPS = _PAIRS // 16
_OUT_W = _PAIRS * _EMBED

_DIV_MAGIC = 5243
_DIV_SHIFT = 20


def _sc_body(q_hbm, key_hbm, tab_hbm, out_hbm,
             tab_v, q_v, key_v, rows0, rows1, sem0, sem1):
    wid = lax.axis_index("s") * _NUM_CORES + lax.axis_index("c")
    pltpu.sync_copy(tab_hbm, tab_v)
    pltpu.sync_copy(q_hbm.at[pl.ds(wid * _ROWS_PER_W, _ROWS_PER_W)], q_v)
    iota = lax.iota(jnp.int32, 16)
    rows = (rows0, rows1)
    sems = (sem0, sem1)

    def out_slice(c):
        return out_hbm.at[pl.ds((wid * _ROWS_PER_W + c * _CB) * _L * _EMBED,
                                _OUT_W)]

    def do_chunk(c, rows_v, sem):
        p0 = (wid * _ROWS_PER_W + c * _CB) * _L
        pltpu.sync_copy(key_hbm.at[pl.ds(p0, _PAIRS)], key_v)
        qbase = c * _CB

        @plsc.parallel_loop(0, _GROUPS, unroll=4)
        def group(v):
            l16 = v * 16 + iota
            b_local = ((l16 * _DIV_MAGIC) >> _DIV_SHIFT) + qbase
            qv = plsc.load_gather(q_v, [b_local])
            kv = key_v[pl.ds(v * 16, 16)]
            idx = jnp.clip(kv - qv, -_BINS, _BINS) + (_BINS + 1)
            a = idx << 4
            o = l16 << 4
            for e in range(_EMBED):
                col = plsc.load_gather(tab_v, [a + e])
                plsc.store_scatter(rows_v, [o + e], col)

        pltpu.async_copy(rows_v, out_slice(c), sem)

    def chunk_pair(c2, carry):
        for h in range(2):
            c = c2 * 2 + h

            @pl.when(c2 > 0)
            def _wait_prev():
                pltpu.make_async_copy(rows[h], out_slice(c - 2), sems[h]).wait()

            do_chunk(c, rows[h], sems[h])
        return carry

    lax.fori_loop(0, _CHUNKS // 2, chunk_pair, 0)
    for h in range(2):
        pltpu.make_async_copy(rows[h], out_slice(_CHUNKS - 2 + h),
                              sems[h]).wait()


@functools.partial(jax.jit, static_argnames=())
def kernel(query_residue_index, key_residue_index, embedding_weight):
    q = query_residue_index.astype(jnp.int32)
    k = key_residue_index.astype(jnp.int32).reshape(_BATCH * _L)
    w = embedding_weight.astype(jnp.float32).reshape(_NUM_EMB * _EMBED)
    f = pl.kernel(
        _sc_body,
        mesh=plsc.VectorSubcoreMesh(core_axis_name="c", subcore_axis_name="s"),
        compiler_params=pltpu.CompilerParams(
            needs_layout_passes=False, use_tc_tiling_on_sc=False
        ),
        out_type=jax.ShapeDtypeStruct((_BATCH * _L * _EMBED,), jnp.float32),
        scratch_types=[
            pltpu.VMEM((_NUM_EMB * _EMBED,), jnp.float32),
            pltpu.VMEM((_ROWS_PER_W,), jnp.int32),
            pltpu.VMEM((_PAIRS,), jnp.int32),
            pltpu.VMEM((_OUT_W,), jnp.float32),
            pltpu.VMEM((_OUT_W,), jnp.float32),
            pltpu.SemaphoreType.DMA,
            pltpu.SemaphoreType.DMA,
        ],
    )
    out = f(q, k, w)
    return out.reshape(_BATCH, _L, _EMBED)


# speedup vs baseline: 6.8035x; 1.4993x over previous
import functools

import jax
import jax.numpy as jnp
from jax import lax
from jax.experimental import pallas as pl
from jax.experimental.pallas import tpu as pltpu
from jax.experimental.pallas import tpu_sc as plsc

_BINS = 32
_EMBED = 16
_BATCH = 16384
_L = 200
_NUM_EMB = 2 * _BINS + 2

_NUM_CORES = 2
_NUM_SUBCORES = 16
_NUM_WORKERS = _NUM_CORES * _NUM_SUBCORES
_ROWS_PER_W = _BATCH // _NUM_WORKERS
_CB = 16
_CHUNKS = _ROWS_PER_W // _CB
_PAIRS = _CB * _L
_GROUPS = _PAIRS // 16
_OUT_W = _PAIRS * _EMBED

_DIV_MAGIC = 5243
_DIV_SHIFT = 20


def _sc_body(q_hbm, key_hbm, tab_hbm, out_hbm,
             tab_v, q_v, key_v, rows0, rows1, sem0, sem1):
    wid = lax.axis_index("s") * _NUM_CORES + lax.axis_index("c")
    pltpu.sync_copy(tab_hbm, tab_v)
    pltpu.sync_copy(q_hbm.at[pl.ds(wid * _ROWS_PER_W, _ROWS_PER_W)], q_v)
    iota = lax.iota(jnp.int32, 16)
    rows = (rows0, rows1)
    sems = (sem0, sem1)

    def out_slice(c):
        return out_hbm.at[pl.ds((wid * _ROWS_PER_W + c * _CB) * _L * _EMBED,
                                _OUT_W)]

    def do_chunk(c, rows_v, sem):
        p0 = (wid * _ROWS_PER_W + c * _CB) * _L
        pltpu.sync_copy(key_hbm.at[pl.ds(p0, _PAIRS)], key_v)
        qbase = c * _CB

        @plsc.parallel_loop(0, _GROUPS, unroll=2)
        def group(v):
            l16 = v * 16 + iota
            b_local = ((l16 * _DIV_MAGIC) >> _DIV_SHIFT) + qbase
            qv = plsc.load_gather(q_v, [b_local])
            kv = key_v[pl.ds(v * 16, 16)]
            a = (jnp.clip(kv - qv, -_BINS, _BINS) + (_BINS + 1)) << 4
            o = v * 256
            for j in range(16):
                row = tab_v[pl.ds(a[j], 16)]
                rows_v[pl.ds(o + j * 16, 16)] = row

        pltpu.async_copy(rows_v, out_slice(c), sem)

    def chunk_pair(c2, carry):
        for h in range(2):
            c = c2 * 2 + h

            @pl.when(c2 > 0)
            def _wait_prev():
                pltpu.make_async_copy(rows[h], out_slice(c - 2), sems[h]).wait()

            do_chunk(c, rows[h], sems[h])
        return carry

    lax.fori_loop(0, _CHUNKS // 2, chunk_pair, 0)
    for h in range(2):
        pltpu.make_async_copy(rows[h], out_slice(_CHUNKS - 2 + h),
                              sems[h]).wait()


@functools.partial(jax.jit, static_argnames=())
def kernel(query_residue_index, key_residue_index, embedding_weight):
    q = query_residue_index.astype(jnp.int32)
    k = key_residue_index.astype(jnp.int32).reshape(_BATCH * _L)
    w = embedding_weight.astype(jnp.float32).reshape(_NUM_EMB * _EMBED)
    f = pl.kernel(
        _sc_body,
        mesh=plsc.VectorSubcoreMesh(core_axis_name="c", subcore_axis_name="s"),
        compiler_params=pltpu.CompilerParams(
            needs_layout_passes=False, use_tc_tiling_on_sc=False
        ),
        out_type=jax.ShapeDtypeStruct((_BATCH * _L * _EMBED,), jnp.float32),
        scratch_types=[
            pltpu.VMEM((_NUM_EMB * _EMBED,), jnp.float32),
            pltpu.VMEM((_ROWS_PER_W,), jnp.int32),
            pltpu.VMEM((_PAIRS,), jnp.int32),
            pltpu.VMEM((_OUT_W,), jnp.float32),
            pltpu.VMEM((_OUT_W,), jnp.float32),
            pltpu.SemaphoreType.DMA,
            pltpu.SemaphoreType.DMA,
        ],
    )
    out = f(q, k, w)
    return out.reshape(_BATCH, _L, _EMBED)
